# Initial kernel scaffold; baseline (speedup 1.0000x reference)
import jax
import jax.numpy as jnp
from jax.experimental import pallas as pl


def kernel(atom_pos, edge_index, dist_exp, atom_embs, emb_table, bilinear_w, bilinear_b, cls_W1, cls_b1, cls_W2, cls_b2, cls_W3, cls_b3, g_W0, g_b0, g_W1, g_b1, g_W2, g_b2, vae_mean_W, vae_mean_b, vae_std_W, vae_std_b, lin_W, lin_b):
    raise NotImplementedError("write your pallas kernel here")


# speedup vs baseline: 1.0028x; 1.0028x over previous
import functools

import jax
import jax.numpy as jnp
from jax.experimental import pallas as pl
from jax.experimental.pallas import tpu as pltpu

N = 10000
DEXP = 128
EMB = 128
HID = 256

_TN = 1000
_FC = 8


def _softplus(x):
    return jnp.logaddexp(x, 0.0)


def _leaky_relu(x):
    return jnp.where(x >= 0, x, 0.01 * x)


def _bilinear_body(d_ref, e_ref, w_ref, o_ref):
    d = d_ref[...]
    e = e_ref[...].astype(jnp.bfloat16)
    acc = jnp.zeros((_TN, HID), jnp.float32)
    for c in range(0, DEXP, _FC):
        parts = [(d[:, f:f + 1].astype(jnp.bfloat16) * e) for f in range(c, c + _FC)]
        o = jnp.concatenate(parts, axis=1)
        acc = acc + jax.lax.dot_general(
            o, w_ref[pl.ds(c * EMB, _FC * EMB), :],
            (((1,), (0,)), ((), ())), preferred_element_type=jnp.float32)
    o_ref[...] = acc


@jax.jit
def _bilinear(dist_exp, emb, w_flat):
    return pl.pallas_call(
        _bilinear_body,
        grid=(N // _TN,),
        in_specs=[
            pl.BlockSpec((_TN, DEXP), lambda i: (i, 0)),
            pl.BlockSpec((_TN, EMB), lambda i: (i, 0)),
            pl.BlockSpec((DEXP * EMB, HID), lambda i: (0, 0)),
        ],
        out_specs=pl.BlockSpec((_TN, HID), lambda i: (i, 0)),
        out_shape=jax.ShapeDtypeStruct((N, HID), jnp.float32),
    )(dist_exp, emb, w_flat)


def kernel(atom_pos, edge_index, dist_exp, atom_embs, emb_table, bilinear_w, bilinear_b,
           cls_W1, cls_b1, cls_W2, cls_b2, cls_W3, cls_b3,
           g_W0, g_b0, g_W1, g_b1, g_W2, g_b2,
           vae_mean_W, vae_mean_b, vae_std_W, vae_std_b, lin_W, lin_b):
    src = edge_index[0]
    dst = edge_index[1]
    n = atom_pos.shape[0]
    emb = emb_table[atom_embs]
    w_flat = bilinear_w.reshape(DEXP * EMB, HID).astype(jnp.bfloat16)
    feat_src = _bilinear(dist_exp, emb, w_flat)
    agg = jnp.zeros((n, HID), feat_src.dtype).at[dst].add(feat_src[src])
    h = agg - feat_src
    rst = _softplus(h) + bilinear_b
    h1 = _softplus(rst @ cls_W1 + cls_b1)
    h2 = _softplus(h1 @ cls_W2 + cls_b2)
    atom_type_pred = _softplus(h2 @ cls_W3 + cls_b3)
    x = emb
    t = atom_type_pred
    loss_atom_pred = jnp.mean(jnp.maximum(x, 0.0) - x * t + jnp.log1p(jnp.exp(-jnp.abs(x))))
    deg_out = jnp.maximum(jnp.bincount(src, length=n), 1).astype(jnp.float32)
    deg_in = jnp.maximum(jnp.bincount(dst, length=n), 1).astype(jnp.float32)
    no = jax.lax.rsqrt(deg_out)[:, None]
    ni = jax.lax.rsqrt(deg_in)[:, None]
    feat = jnp.concatenate([atom_pos, emb], axis=-1)
    for W, b in ((g_W0, g_b0), (g_W1, g_b1), (g_W2, g_b2)):
        fs = feat * no
        ag = jnp.zeros((n, fs.shape[1]), fs.dtype).at[dst].add(fs[src])
        feat = _softplus((ag * ni) @ W + b)
    mean = _leaky_relu(feat @ vae_mean_W + vae_mean_b)
    logstd = _leaky_relu(feat @ vae_std_W + vae_std_b)
    atom_pos_vae = mean + jnp.exp(0.5 * logstd)
    loss_vae = 0.5 * jnp.sum(1.0 + logstd - jnp.square(mean) - jnp.exp(logstd))
    atom_pos_pred = atom_pos_vae @ lin_W + lin_b
    loss_pos_pred = jnp.mean(jnp.square(atom_pos - atom_pos_pred))
    return (loss_atom_pred, loss_pos_pred, loss_vae)
